# baseline (speedup 1.0000x reference)
import jax
import jax.numpy as jnp
from jax import lax
from jax.experimental import pallas as pl
from jax.experimental.pallas import tpu as pltpu
from jax.experimental.pallas import tpu_sc as plsc

VOCAB_PAD = 1024

NC = 2
NS = 16
NW = NC * NS

R = 16384
C = 200
PER_W = R // NW
RBLK = 64
NCHUNK = PER_W // RBLK
L = 16

COLS = tuple(range(0, C - L + 1, L)) + ((C - L),)


def _body(probs_hbm, alias_hbm, index_hbm, prob_hbm, out_hbm,
          probs_v, alias_v, idx_v0, idx_v1, prob_v0, prob_v1, out_v0, out_v1,
          si0, si1, sp0, sp1, so0, so1):
    wid = lax.axis_index("s") * NC + lax.axis_index("c")
    base = wid * PER_W

    pltpu.sync_copy(probs_hbm, probs_v)
    pltpu.sync_copy(alias_hbm, alias_v)

    si = (si0, si1)
    sp = (sp0, sp1)
    so = (so0, so1)
    idx_b = (idx_v0, idx_v1)
    prob_b = (prob_v0, prob_v1)
    out_b = (out_v0, out_v1)
    in_desc = [None, None]
    out_desc = [None, None]

    in_desc[0] = (
        pltpu.async_copy(index_hbm.at[pl.ds(base, RBLK), :], idx_b[0], si[0]),
        pltpu.async_copy(prob_hbm.at[pl.ds(base, RBLK), :], prob_b[0], sp[0]),
    )

    for j in range(NCHUNK):
        buf = j % 2
        nxt = 1 - buf
        if j + 1 < NCHUNK:
            rn = base + (j + 1) * RBLK
            in_desc[nxt] = (
                pltpu.async_copy(index_hbm.at[pl.ds(rn, RBLK), :],
                                 idx_b[nxt], si[nxt]),
                pltpu.async_copy(prob_hbm.at[pl.ds(rn, RBLK), :],
                                 prob_b[nxt], sp[nxt]),
            )
        di, dp = in_desc[buf]
        di.wait()
        dp.wait()
        if out_desc[buf] is not None:
            out_desc[buf].wait()

        ib = idx_b[buf]
        pb = prob_b[buf]
        ob = out_b[buf]

        def inner(r, _):
            for c in COLS:
                s = pl.ds(c, L)
                idx = ib[r, s]
                pv = pb[r, s]
                pa = plsc.load_gather(probs_v, [idx])
                al = plsc.load_gather(alias_v, [idx])
                ob[r, s] = jnp.where(pv < pa, idx, al)
            return 0

        lax.fori_loop(0, RBLK, inner, 0)
        out_desc[buf] = pltpu.async_copy(
            ob, out_hbm.at[pl.ds(base + j * RBLK, RBLK), :], so[buf])

    out_desc[0].wait()
    out_desc[1].wait()


@jax.jit
def _sample(probs_pad, alias_pad, index, prob):
    mesh = plsc.VectorSubcoreMesh(core_axis_name="c", subcore_axis_name="s")
    return pl.kernel(
        _body,
        out_type=jax.ShapeDtypeStruct((R, C), jnp.int32),
        mesh=mesh,
        scratch_types=[
            pltpu.VMEM((VOCAB_PAD,), jnp.float32),
            pltpu.VMEM((VOCAB_PAD,), jnp.int32),
            pltpu.VMEM((RBLK, C), jnp.int32),
            pltpu.VMEM((RBLK, C), jnp.int32),
            pltpu.VMEM((RBLK, C), jnp.float32),
            pltpu.VMEM((RBLK, C), jnp.float32),
            pltpu.VMEM((RBLK, C), jnp.int32),
            pltpu.VMEM((RBLK, C), jnp.int32),
            pltpu.SemaphoreType.DMA,
            pltpu.SemaphoreType.DMA,
            pltpu.SemaphoreType.DMA,
            pltpu.SemaphoreType.DMA,
            pltpu.SemaphoreType.DMA,
            pltpu.SemaphoreType.DMA,
        ],
        compiler_params=pltpu.CompilerParams(needs_layout_passes=False,
                                             use_tc_tiling_on_sc=True),
    )(probs_pad, alias_pad, index, prob)


def kernel(probs, alias, index, prob):
    v = probs.shape[0]
    probs_pad = jnp.pad(probs, (0, VOCAB_PAD - v))
    alias_pad = jnp.pad(alias, (0, VOCAB_PAD - v))
    return _sample(probs_pad, alias_pad, index, prob)


# speedup vs baseline: 1.4248x; 1.4248x over previous
import jax
import jax.numpy as jnp
from jax import lax
from jax.experimental import pallas as pl
from jax.experimental.pallas import tpu as pltpu
from jax.experimental.pallas import tpu_sc as plsc

VOCAB_PAD = 1024

NC = 2
NS = 16
NW = NC * NS

R = 200
CT = 16384
CPW = CT // NW
RBLK = 8
NCHUNK = R // RBLK
NPAIR = (NCHUNK - 1) // 2
L = 16
NSLICE = CPW // L


def _body(probs_hbm, alias_hbm, index_hbm, prob_hbm, out_hbm,
          probs_v, alias_v, idx_v0, idx_v1, prob_v0, prob_v1, out_v0, out_v1,
          si0, si1, sp0, sp1, so0, so1):
    wid = lax.axis_index("s") * NC + lax.axis_index("c")
    c0 = wid * CPW

    pltpu.sync_copy(probs_hbm, probs_v)
    pltpu.sync_copy(alias_hbm, alias_v)

    si = (si0, si1)
    sp = (sp0, sp1)
    so = (so0, so1)
    idx_b = (idx_v0, idx_v1)
    prob_b = (prob_v0, prob_v1)
    out_b = (out_v0, out_v1)

    def issue_in(j, b):
        rn = j * RBLK
        pltpu.async_copy(index_hbm.at[pl.ds(rn, RBLK), pl.ds(c0, CPW)],
                         idx_b[b], si[b])
        pltpu.async_copy(prob_hbm.at[pl.ds(rn, RBLK), pl.ds(c0, CPW)],
                         prob_b[b], sp[b])

    def wait_in(b):
        pltpu.make_async_copy(index_hbm.at[pl.ds(0, RBLK), pl.ds(c0, CPW)],
                              idx_b[b], si[b]).wait()
        pltpu.make_async_copy(prob_hbm.at[pl.ds(0, RBLK), pl.ds(c0, CPW)],
                              prob_b[b], sp[b]).wait()

    def issue_out(j, b):
        pltpu.async_copy(out_b[b],
                         out_hbm.at[pl.ds(j * RBLK, RBLK), pl.ds(c0, CPW)],
                         so[b])

    def wait_out(b):
        pltpu.make_async_copy(out_b[b],
                              out_hbm.at[pl.ds(0, RBLK), pl.ds(c0, CPW)],
                              so[b]).wait()

    def compute(b):
        ib = idx_b[b]
        pb = prob_b[b]
        ob = out_b[b]

        def inner(c, _):
            s = pl.ds(c * L, L)
            for r in range(RBLK):
                idx = ib[r, s]
                pv = pb[r, s]
                pa = plsc.load_gather(probs_v, [idx])
                al = plsc.load_gather(alias_v, [idx])
                ob[r, s] = jnp.where(pv < pa, idx, al)
            return 0

        lax.fori_loop(0, NSLICE, inner, 0)

    issue_in(0, 0)
    issue_in(1, 1)

    def pair(k, _):
        j0 = k * 2
        for b in range(2):
            j = j0 + b
            wait_in(b)

            @pl.when(k > 0)
            def _():
                wait_out(b)

            compute(b)
            issue_out(j, b)

            @pl.when(j + 2 < NCHUNK)
            def _():
                issue_in(j + 2, b)
        return 0

    lax.fori_loop(0, NPAIR, pair, 0)

    wait_in(0)
    wait_out(0)
    compute(0)
    issue_out(NCHUNK - 1, 0)

    wait_out(1)
    wait_out(0)


@jax.jit
def _sample(probs_pad, alias_pad, index_t, prob_t):
    mesh = plsc.VectorSubcoreMesh(core_axis_name="c", subcore_axis_name="s")
    return pl.kernel(
        _body,
        out_type=jax.ShapeDtypeStruct((R, CT), jnp.int32),
        mesh=mesh,
        scratch_types=[
            pltpu.VMEM((VOCAB_PAD,), jnp.float32),
            pltpu.VMEM((VOCAB_PAD,), jnp.int32),
            pltpu.VMEM((RBLK, CPW), jnp.int32),
            pltpu.VMEM((RBLK, CPW), jnp.int32),
            pltpu.VMEM((RBLK, CPW), jnp.float32),
            pltpu.VMEM((RBLK, CPW), jnp.float32),
            pltpu.VMEM((RBLK, CPW), jnp.int32),
            pltpu.VMEM((RBLK, CPW), jnp.int32),
            pltpu.SemaphoreType.DMA,
            pltpu.SemaphoreType.DMA,
            pltpu.SemaphoreType.DMA,
            pltpu.SemaphoreType.DMA,
            pltpu.SemaphoreType.DMA,
            pltpu.SemaphoreType.DMA,
        ],
        compiler_params=pltpu.CompilerParams(needs_layout_passes=False),
    )(probs_pad, alias_pad, index_t, prob_t)


def kernel(probs, alias, index, prob):
    v = probs.shape[0]
    probs_pad = jnp.pad(probs, (0, VOCAB_PAD - v))
    alias_pad = jnp.pad(alias, (0, VOCAB_PAD - v))
    out_t = _sample(probs_pad, alias_pad,
                    jnp.swapaxes(index, 0, 1), jnp.swapaxes(prob, 0, 1))
    return jnp.swapaxes(out_t, 0, 1)


# speedup vs baseline: 2.3134x; 1.6237x over previous
import jax
import jax.numpy as jnp
from jax import lax
from jax.experimental import pallas as pl
from jax.experimental.pallas import tpu as pltpu
from jax.experimental.pallas import tpu_sc as plsc

VOCAB_PAD = 1024

NC = 2
NS = 16
NW = NC * NS

R = 200
CT = 16384
CPW = CT // NW
RBLK = 8
NCHUNK = R // RBLK
NPAIR = (NCHUNK - 1) // 2
L = 16
NSLICE = CPW // L


def _body(probs_hbm, alias_hbm, index_hbm, prob_hbm, out_hbm,
          probs_v, alias_v, idx_v0, idx_v1, prob_v0, prob_v1, out_v0, out_v1,
          si0, si1, sp0, sp1, so0, so1):
    wid = lax.axis_index("s") * NC + lax.axis_index("c")
    c0 = wid * CPW

    pltpu.sync_copy(probs_hbm, probs_v)
    pltpu.sync_copy(alias_hbm, alias_v)

    si = (si0, si1)
    sp = (sp0, sp1)
    so = (so0, so1)
    idx_b = (idx_v0, idx_v1)
    prob_b = (prob_v0, prob_v1)
    out_b = (out_v0, out_v1)

    def issue_in(j, b):
        rn = j * RBLK
        pltpu.async_copy(index_hbm.at[pl.ds(rn, RBLK), pl.ds(c0, CPW)],
                         idx_b[b], si[b])
        pltpu.async_copy(prob_hbm.at[pl.ds(rn, RBLK), pl.ds(c0, CPW)],
                         prob_b[b], sp[b])

    def wait_in(b):
        pltpu.make_async_copy(index_hbm.at[pl.ds(0, RBLK), pl.ds(c0, CPW)],
                              idx_b[b], si[b]).wait()
        pltpu.make_async_copy(prob_hbm.at[pl.ds(0, RBLK), pl.ds(c0, CPW)],
                              prob_b[b], sp[b]).wait()

    def issue_out(j, b):
        pltpu.async_copy(out_b[b],
                         out_hbm.at[pl.ds(j * RBLK, RBLK), pl.ds(c0, CPW)],
                         so[b])

    def wait_out(b):
        pltpu.make_async_copy(out_b[b],
                              out_hbm.at[pl.ds(0, RBLK), pl.ds(c0, CPW)],
                              so[b]).wait()

    def compute(b):
        ib = idx_b[b]
        pb = prob_b[b]
        ob = out_b[b]

        @plsc.parallel_loop(0, NSLICE, 1, unroll=2)
        def _(c):
            s = pl.ds(c * L, L)
            for r in range(RBLK):
                idx = ib[r, s]
                pv = pb[r, s]
                pa = plsc.load_gather(probs_v, [idx])
                al = plsc.load_gather(alias_v, [idx])
                ob[r, s] = jnp.where(pv < pa, idx, al)

    issue_in(0, 0)
    issue_in(1, 1)

    def pair(k, _):
        j0 = k * 2
        for b in range(2):
            j = j0 + b
            wait_in(b)

            @pl.when(k > 0)
            def _():
                wait_out(b)

            compute(b)
            issue_out(j, b)

            @pl.when(j + 2 < NCHUNK)
            def _():
                issue_in(j + 2, b)
        return 0

    lax.fori_loop(0, NPAIR, pair, 0)

    wait_in(0)
    wait_out(0)
    compute(0)
    issue_out(NCHUNK - 1, 0)

    wait_out(1)
    wait_out(0)


@jax.jit
def _sample(probs_pad, alias_pad, index_t, prob_t):
    mesh = plsc.VectorSubcoreMesh(core_axis_name="c", subcore_axis_name="s")
    return pl.kernel(
        _body,
        out_type=jax.ShapeDtypeStruct((R, CT), jnp.int32),
        mesh=mesh,
        scratch_types=[
            pltpu.VMEM((VOCAB_PAD,), jnp.float32),
            pltpu.VMEM((VOCAB_PAD,), jnp.int32),
            pltpu.VMEM((RBLK, CPW), jnp.int32),
            pltpu.VMEM((RBLK, CPW), jnp.int32),
            pltpu.VMEM((RBLK, CPW), jnp.float32),
            pltpu.VMEM((RBLK, CPW), jnp.float32),
            pltpu.VMEM((RBLK, CPW), jnp.int32),
            pltpu.VMEM((RBLK, CPW), jnp.int32),
            pltpu.SemaphoreType.DMA,
            pltpu.SemaphoreType.DMA,
            pltpu.SemaphoreType.DMA,
            pltpu.SemaphoreType.DMA,
            pltpu.SemaphoreType.DMA,
            pltpu.SemaphoreType.DMA,
        ],
        compiler_params=pltpu.CompilerParams(needs_layout_passes=False),
    )(probs_pad, alias_pad, index_t, prob_t)


def kernel(probs, alias, index, prob):
    v = probs.shape[0]
    probs_pad = jnp.pad(probs, (0, VOCAB_PAD - v))
    alias_pad = jnp.pad(alias, (0, VOCAB_PAD - v))
    out_t = _sample(probs_pad, alias_pad,
                    jnp.swapaxes(index, 0, 1), jnp.swapaxes(prob, 0, 1))
    return jnp.swapaxes(out_t, 0, 1)
